# speedup vs baseline: 2.9746x; 2.9746x over previous
import functools

import jax
import jax.numpy as jnp
from jax import lax
from jax.experimental import pallas as pl
from jax.experimental.pallas import tpu as pltpu
from jax.experimental.pallas import tpu_sc as plsc

_VOCAB = 100000
_EMB = 128
_B = 4096
_L = 50

_NC = 2
_NS = 16
_NW = _NC * _NS
_N = _B * _L
_PER_W = _N // _NW
_CHUNK = 128
_NCHUNK = _PER_W // _CHUNK

_mesh = plsc.VectorSubcoreMesh(core_axis_name="c", subcore_axis_name="s")


@functools.partial(
    pl.kernel,
    out_type=jax.ShapeDtypeStruct((_N, _EMB), jnp.float32),
    mesh=_mesh,
    scratch_types=[
        pltpu.VMEM((_NCHUNK, _CHUNK), jnp.int32),
        pltpu.VMEM((_CHUNK, _EMB), jnp.float32),
        pltpu.SemaphoreType.DMA,
    ],
)
def _emb_lookup(idx_hbm, table_hbm, out_hbm, idx_v, rows_v, gsem):
    wid = lax.axis_index("s") * _NC + lax.axis_index("c")
    base = wid * _PER_W
    pltpu.sync_copy(idx_hbm.at[wid], idx_v)

    def body(j, carry):
        pltpu.async_copy(table_hbm.at[idx_v.at[j]], rows_v, gsem).wait()
        pltpu.sync_copy(rows_v, out_hbm.at[pl.ds(base + j * _CHUNK, _CHUNK)])
        return carry

    lax.fori_loop(0, _NCHUNK, body, 0)


def kernel(input, table):
    idx = input.reshape(_NW, _NCHUNK, _CHUNK).astype(jnp.int32)
    out = _emb_lookup(idx, table)
    return out.reshape(_B, _L, _EMB)


# speedup vs baseline: 3.3458x; 1.1248x over previous
import functools

import jax
import jax.numpy as jnp
from jax import lax
from jax.experimental import pallas as pl
from jax.experimental.pallas import tpu as pltpu
from jax.experimental.pallas import tpu_sc as plsc

_VOCAB = 100000
_EMB = 128
_B = 4096
_L = 50

_NC = 2
_NS = 16
_NW = _NC * _NS
_N = _B * _L
_PER_W = _N // _NW
_CHUNK = 128
_NCHUNK = _PER_W // _CHUNK
_NBUF = 2

_mesh = plsc.VectorSubcoreMesh(core_axis_name="c", subcore_axis_name="s")


@functools.partial(
    pl.kernel,
    out_type=jax.ShapeDtypeStruct((_N, _EMB), jnp.float32),
    mesh=_mesh,
    scratch_types=[
        pltpu.VMEM((_NCHUNK, _CHUNK), jnp.int32),
        pltpu.VMEM((_NBUF, _CHUNK, _EMB), jnp.float32),
        pltpu.SemaphoreType.DMA,
        pltpu.SemaphoreType.DMA,
        pltpu.SemaphoreType.DMA,
        pltpu.SemaphoreType.DMA,
    ],
)
def _emb_lookup(idx_hbm, table_hbm, out_hbm, idx_v, rows_v, g0, g1, s0, s1):
    wid = lax.axis_index("s") * _NC + lax.axis_index("c")
    base = wid * _PER_W
    gsem = (g0, g1)
    ssem = (s0, s1)
    pltpu.sync_copy(idx_hbm.at[wid], idx_v)

    def start_gather(j, b):
        pltpu.make_async_copy(table_hbm.at[idx_v.at[j]], rows_v.at[b], gsem[b]).start()

    def wait_gather(b):
        pltpu.make_async_copy(table_hbm.at[pl.ds(0, _CHUNK)], rows_v.at[b], gsem[b]).wait()

    def store(j, b):
        return pltpu.make_async_copy(
            rows_v.at[b], out_hbm.at[pl.ds(base + j * _CHUNK, _CHUNK)], ssem[b]
        )

    for b in range(_NBUF):
        start_gather(b, b)

    def body(i, carry):
        j0 = i * _NBUF
        for b in range(_NBUF):
            j = j0 + b
            wait_gather(b)
            store(j, b).start()
            store(j, b).wait()
            start_gather(j + _NBUF, b)
        return carry

    lax.fori_loop(0, (_NCHUNK - _NBUF) // _NBUF, body, 0)

    for b in range(_NBUF):
        j = _NCHUNK - _NBUF + b
        wait_gather(b)
        pltpu.sync_copy(rows_v.at[b], out_hbm.at[pl.ds(base + j * _CHUNK, _CHUNK)])


def kernel(input, table):
    idx = input.reshape(_NW, _NCHUNK, _CHUNK).astype(jnp.int32)
    out = _emb_lookup(idx, table)
    return out.reshape(_B, _L, _EMB)
